# baseline (speedup 1.0000x reference)
import functools

import jax
import jax.numpy as jnp
from jax import lax
from jax.experimental import pallas as pl
from jax.experimental.pallas import tpu as pltpu
from jax.experimental.pallas import tpu_sc as plsc

MAXLEN = 8192
EMBED_DIM = 128
REPEATS = 4
OUT_ROWS = MAXLEN * REPEATS

NUM_CORES = 2
HALF_ROWS = MAXLEN // NUM_CORES
N_CHUNKS = 8
CHUNK_ROWS = HALF_ROWS // N_CHUNKS


@functools.partial(
    pl.kernel,
    mesh=plsc.ScalarSubcoreMesh(axis_name="c", num_cores=2),
    out_type=jax.ShapeDtypeStruct((MAXLEN, REPEATS, EMBED_DIM), jnp.float32),
    scratch_types=[
        pltpu.VMEM_SHARED((N_CHUNKS, CHUNK_ROWS, EMBED_DIM), jnp.float32),
        pltpu.SemaphoreType.DMA,
        pltpu.SemaphoreType.DMA,
    ],
)
def _pos_embed_scs(table_hbm, out_hbm, buf, ld_sem, st_sem):
    base = lax.axis_index("c") * HALF_ROWS
    loads = [
        pltpu.make_async_copy(
            table_hbm.at[pl.ds(base + k * CHUNK_ROWS, CHUNK_ROWS)],
            buf.at[k],
            ld_sem,
        )
        for k in range(N_CHUNKS)
    ]
    for ld in loads:
        ld.start()
    stores = []
    for k in range(N_CHUNKS):
        loads[k].wait()
        for j in range(REPEATS):
            c = pltpu.make_async_copy(
                buf.at[k],
                out_hbm.at[pl.ds(base + k * CHUNK_ROWS, CHUNK_ROWS), j],
                st_sem,
            )
            c.start()
            stores.append(c)
    for c in stores:
        c.wait()


def kernel(inputs, pos_emb):
    out3 = _pos_embed_scs(pos_emb)
    return out3.reshape(OUT_ROWS, EMBED_DIM)


# speedup vs baseline: 1.1021x; 1.1021x over previous
import functools

import jax
import jax.numpy as jnp
from jax import lax
from jax.experimental import pallas as pl
from jax.experimental.pallas import tpu as pltpu
from jax.experimental.pallas import tpu_sc as plsc

MAXLEN = 8192
EMBED_DIM = 128
REPEATS = 4
OUT_ROWS = MAXLEN * REPEATS

NUM_CORES = 2
NUM_SUBCORES = 16
NUM_WORKERS = NUM_CORES * NUM_SUBCORES
ROWS_PER_WORKER = MAXLEN // NUM_WORKERS


N_CHUNKS = 4
CHUNK_ROWS = ROWS_PER_WORKER // N_CHUNKS


@functools.partial(
    pl.kernel,
    mesh=plsc.VectorSubcoreMesh(core_axis_name="c", subcore_axis_name="s"),
    out_type=jax.ShapeDtypeStruct((MAXLEN, REPEATS, EMBED_DIM), jnp.float32),
    scratch_types=[
        pltpu.VMEM((N_CHUNKS, CHUNK_ROWS, EMBED_DIM), jnp.float32),
        pltpu.SemaphoreType.DMA,
        pltpu.SemaphoreType.DMA,
    ],
)
def _pos_embed_sc(table_hbm, out_hbm, rows_v, ld_sem, st_sem):
    wid = lax.axis_index("c") * NUM_SUBCORES + lax.axis_index("s")
    base = wid * ROWS_PER_WORKER
    loads = [
        pltpu.make_async_copy(
            table_hbm.at[pl.ds(base + k * CHUNK_ROWS, CHUNK_ROWS)],
            rows_v.at[k],
            ld_sem,
        )
        for k in range(N_CHUNKS)
    ]
    for ld in loads:
        ld.start()
    stores = []
    for k in range(N_CHUNKS):
        loads[k].wait()
        for j in range(REPEATS):
            c = pltpu.make_async_copy(
                rows_v.at[k],
                out_hbm.at[pl.ds(base + k * CHUNK_ROWS, CHUNK_ROWS), j],
                st_sem,
            )
            c.start()
            stores.append(c)
    for c in stores:
        c.wait()


def kernel(inputs, pos_emb):
    out3 = _pos_embed_sc(pos_emb)
    return out3.reshape(OUT_ROWS, EMBED_DIM)
